# Initial kernel scaffold; baseline (speedup 1.0000x reference)
import jax
import jax.numpy as jnp
from jax.experimental import pallas as pl


def kernel(xs, table):
    raise NotImplementedError("write your pallas kernel here")


# speedup vs baseline: 2.9709x; 2.9709x over previous
import functools

import jax
import jax.numpy as jnp
from jax import lax
from jax.experimental import pallas as pl
from jax.experimental.pallas import tpu as pltpu
from jax.experimental.pallas import tpu_sc as plsc

NC = 2
NS = 16
NW = NC * NS
R = 128
D = 128


@functools.cache
def _gather_kernel(total_rows: int):
  rows_per_w = total_rows // NW
  steps = rows_per_w // R
  mesh = plsc.VectorSubcoreMesh(core_axis_name="c", subcore_axis_name="s")

  @functools.partial(
      pl.kernel,
      out_type=jax.ShapeDtypeStruct((total_rows, D), jnp.float32),
      mesh=mesh,
      scratch_types=[
          pltpu.VMEM((rows_per_w,), jnp.int32),
          pltpu.VMEM((R, D), jnp.float32),
          pltpu.SemaphoreType.DMA,
      ],
  )
  def k(idx_hbm, table_hbm, out_hbm, idx_v, rows_v, sem):
    wid = lax.axis_index("s") * NC + lax.axis_index("c")
    base = wid * rows_per_w
    pltpu.sync_copy(idx_hbm.at[pl.ds(base, rows_per_w)], idx_v)

    def body(j, carry):
      pltpu.async_copy(
          table_hbm.at[idx_v.at[pl.ds(j * R, R)]], rows_v, sem
      ).wait()
      pltpu.sync_copy(rows_v, out_hbm.at[pl.ds(base + j * R, R)])
      return carry

    lax.fori_loop(0, steps, body, 0)

  return k


def kernel(xs, table):
  b, s = xs.shape
  total = b * s
  idx = xs.reshape(total).astype(jnp.int32)
  out = _gather_kernel(total)(idx, table)
  return out.reshape(b, s, D)


# speedup vs baseline: 3.3290x; 1.1205x over previous
import functools

import jax
import jax.numpy as jnp
from jax import lax
from jax.experimental import pallas as pl
from jax.experimental.pallas import tpu as pltpu
from jax.experimental.pallas import tpu_sc as plsc

NC = 2
NS = 16
NW = NC * NS
R = 128
D = 128
NSLOT = 5
K = 3


@functools.cache
def _gather_kernel(total_rows: int):
  rows_per_w = total_rows // NW
  steps = rows_per_w // R
  nouter = steps // NSLOT
  assert steps % NSLOT == 0 and K < NSLOT and nouter >= 2
  mesh = plsc.VectorSubcoreMesh(core_axis_name="c", subcore_axis_name="s")

  @functools.partial(
      pl.kernel,
      out_type=jax.ShapeDtypeStruct((total_rows, D), jnp.float32),
      mesh=mesh,
      scratch_types=[
          pltpu.VMEM((rows_per_w,), jnp.int32),
          pltpu.VMEM((NSLOT, R, D), jnp.float32),
          pltpu.SemaphoreType.DMA((NSLOT,)),
          pltpu.SemaphoreType.DMA((NSLOT,)),
      ],
  )
  def k(idx_hbm, table_hbm, out_hbm, idx_v, bufs, gsem, wsem):
    wid = lax.axis_index("s") * NC + lax.axis_index("c")
    base = wid * rows_per_w
    pltpu.sync_copy(idx_hbm.at[pl.ds(base, rows_per_w)], idx_v)

    def gstart(j, b):
      pltpu.async_copy(
          table_hbm.at[idx_v.at[pl.ds(j * R, R)]], bufs.at[b], gsem.at[b]
      )

    def gwait(j, b):
      pltpu.make_async_copy(
          table_hbm.at[idx_v.at[pl.ds(j * R, R)]], bufs.at[b], gsem.at[b]
      ).wait()

    def wstart(j, b):
      pltpu.async_copy(
          bufs.at[b], out_hbm.at[pl.ds(base + j * R, R)], wsem.at[b]
      )

    def wwait(b):
      pltpu.make_async_copy(
          bufs.at[b], out_hbm.at[pl.ds(base, R)], wsem.at[b]
      ).wait()

    for m in range(K):
      gstart(m, m)

    for b in range(NSLOT):
      sb = (b + K) % NSLOT
      if b + K >= NSLOT:
        wwait(sb)
      gstart(b + K, sb)
      gwait(b, b)
      wstart(b, b)

    def outer(g, carry):
      j0 = g * NSLOT
      for b in range(NSLOT):
        j = j0 + b
        sb = (b + K) % NSLOT
        wwait(sb)
        gstart(j + K, sb)
        gwait(j, b)
        wstart(j, b)
      return carry

    lax.fori_loop(1, nouter - 1, outer, 0)

    j0 = (nouter - 1) * NSLOT
    for b in range(NSLOT):
      j = j0 + b
      sb = (b + K) % NSLOT
      if j + K < steps:
        wwait(sb)
        gstart(j + K, sb)
      gwait(j, b)
      wstart(j, b)

    for b in range(NSLOT):
      wwait(b)

  return k


def kernel(xs, table):
  b, s = xs.shape
  total = b * s
  idx = xs.reshape(total).astype(jnp.int32)
  out = _gather_kernel(total)(idx, table)
  return out.reshape(b, s, D)


# speedup vs baseline: 5.9751x; 1.7949x over previous
import functools

import jax
import jax.numpy as jnp
from jax import lax
from jax.experimental import pallas as pl
from jax.experimental.pallas import tpu as pltpu
from jax.experimental.pallas import tpu_sc as plsc

NC = 2
NS = 16
NW = NC * NS
D = 128
SP = 56
EC = 2
NSLOT = 8
K = 6


@functools.cache
def _gather_kernel(batch: int, seq: int):
  entries_per_w = batch // NW
  steps = entries_per_w // EC
  nouter = steps // NSLOT
  idx_per_w = entries_per_w * SP
  assert steps % NSLOT == 0 and K < NSLOT and nouter >= 2
  mesh = plsc.VectorSubcoreMesh(core_axis_name="c", subcore_axis_name="s")

  @functools.partial(
      pl.kernel,
      out_type=jax.ShapeDtypeStruct((batch, seq, D), jnp.float32),
      mesh=mesh,
      scratch_types=[
          pltpu.VMEM((idx_per_w,), jnp.int32),
          pltpu.VMEM((NSLOT, EC, seq, D), jnp.float32),
          pltpu.SemaphoreType.DMA((NSLOT,)),
          pltpu.SemaphoreType.DMA((NSLOT,)),
      ],
  )
  def k(idx_hbm, table_hbm, out_hbm, idx_v, bufs, gsem, wsem):
    wid = lax.axis_index("s") * NC + lax.axis_index("c")
    e0 = wid * entries_per_w
    pltpu.sync_copy(idx_hbm.at[pl.ds(wid * idx_per_w, idx_per_w)], idx_v)

    def gstart(ch, b):
      for e in range(EC):
        pltpu.async_copy(
            table_hbm.at[idx_v.at[pl.ds((ch * EC + e) * SP, seq)]],
            bufs.at[b, e],
            gsem.at[b],
        )

    def gwait(b):
      for e in range(EC):
        pltpu.make_async_copy(
            table_hbm.at[idx_v.at[pl.ds(0, seq)]], bufs.at[b, e], gsem.at[b]
        ).wait()

    def wstart(ch, b):
      pltpu.async_copy(
          bufs.at[b], out_hbm.at[pl.ds(e0 + ch * EC, EC)], wsem.at[b]
      )

    def wwait(b):
      pltpu.make_async_copy(
          bufs.at[b], out_hbm.at[pl.ds(e0, EC)], wsem.at[b]
      ).wait()

    for m in range(K):
      gstart(m, m)

    for b in range(NSLOT):
      sb = (b + K) % NSLOT
      if b + K >= NSLOT:
        wwait(sb)
      gstart(b + K, sb)
      gwait(b)
      wstart(b, b)

    def outer(g, carry):
      ch0 = g * NSLOT
      for b in range(NSLOT):
        ch = ch0 + b
        sb = (b + K) % NSLOT
        wwait(sb)
        gstart(ch + K, sb)
        gwait(b)
        wstart(ch, b)
      return carry

    lax.fori_loop(1, nouter - 1, outer, 0)

    ch0 = (nouter - 1) * NSLOT
    for b in range(NSLOT):
      ch = ch0 + b
      sb = (b + K) % NSLOT
      if ch + K < steps:
        wwait(sb)
        gstart(ch + K, sb)
      gwait(b)
      wstart(ch, b)

    for b in range(NSLOT):
      wwait(b)

  return k


def kernel(xs, table):
  b, s = xs.shape
  idx = jnp.pad(xs.astype(jnp.int32), ((0, 0), (0, SP - s))).reshape(b * SP)
  return _gather_kernel(b, s)(idx, table)


# speedup vs baseline: 10.4443x; 1.7480x over previous
import functools

import jax
import jax.numpy as jnp
from jax import lax
from jax.experimental import pallas as pl
from jax.experimental.pallas import tpu as pltpu
from jax.experimental.pallas import tpu_sc as plsc

NC = 2
NS = 16
NW = NC * NS
R = 128
D = 128
NSLOT = 5
K = 3


@functools.cache
def _gather_kernel(total_rows: int):
  rows_per_w = total_rows // NW
  steps = rows_per_w // R
  nouter = steps // NSLOT
  assert steps % NSLOT == 0 and K < NSLOT and nouter >= 2
  mesh = plsc.VectorSubcoreMesh(core_axis_name="c", subcore_axis_name="s")

  @functools.partial(
      pl.kernel,
      out_type=jax.ShapeDtypeStruct((total_rows, D), jnp.float32),
      mesh=mesh,
      scratch_types=[
          pltpu.VMEM((rows_per_w,), jnp.int32),
          pltpu.VMEM((NSLOT, R, D), jnp.float32),
          pltpu.SemaphoreType.DMA((NSLOT,)),
          pltpu.SemaphoreType.DMA((NSLOT,)),
      ],
  )
  def k(idx_hbm, table_hbm, out_hbm, idx_v, bufs, gsem, wsem):
    wid = lax.axis_index("s") * NC + lax.axis_index("c")
    base = wid * rows_per_w
    pltpu.sync_copy(idx_hbm.at[pl.ds(base, rows_per_w)], idx_v)

    def gstart(j, b):
      pltpu.async_copy(
          table_hbm.at[idx_v.at[pl.ds(j * R, R)]], bufs.at[b], gsem.at[b]
      )

    def gwait(j, b):
      pltpu.make_async_copy(
          table_hbm.at[idx_v.at[pl.ds(j * R, R)]], bufs.at[b], gsem.at[b]
      ).wait()

    def wstart(j, b):
      pltpu.async_copy(
          bufs.at[b], out_hbm.at[pl.ds(base + j * R, R)], wsem.at[b]
      )

    def wwait(b):
      pltpu.make_async_copy(
          bufs.at[b], out_hbm.at[pl.ds(base, R)], wsem.at[b]
      ).wait()

    for m in range(K):
      gstart(m, m)

    for b in range(NSLOT):
      sb = (b + K) % NSLOT
      if b + K >= NSLOT:
        wwait(sb)
      gstart(b + K, sb)
      gwait(b, b)
      wstart(b, b)

    def outer(g, carry):
      j0 = g * NSLOT
      for b in range(NSLOT):
        j = j0 + b
        sb = (b + K) % NSLOT
        wwait(sb)
        gstart(j + K, sb)
        gwait(j, b)
        wstart(j, b)
      return carry

    lax.fori_loop(1, nouter - 1, outer, 0)

    j0 = (nouter - 1) * NSLOT
    for b in range(NSLOT):
      j = j0 + b
      sb = (b + K) % NSLOT
      if j + K < steps:
        wwait(sb)
        gstart(j + K, sb)
      gwait(j, b)
      wstart(j, b)

    for b in range(NSLOT):
      wwait(b)

  return k


def kernel(xs, table):
  b, s = xs.shape
  total = b * s
  idx = xs.astype(jnp.int32).T.reshape(total)
  out = _gather_kernel(total)(idx, table)
  return out.reshape(s, b, D).transpose(1, 0, 2)
